# baseline (speedup 1.0000x reference)
import functools

import jax
import jax.numpy as jnp
import numpy as np
from jax import lax
from jax.experimental import pallas as pl
from jax.experimental.pallas import tpu as pltpu
from jax.experimental.pallas import tpu_sc as plsc

N = 10000
E = 320000
IN_C = 128
HID_C = 16
OUT_C = 8
H1 = 8
HC = H1 * HID_C
ROWW = HC + 16
ROWW2 = 16

NCORES = 2
NSUB = 16
NTILES = NCORES * NSUB
EPT = E // NTILES
CHUNK = 40
STAGE = 2000
NSTAGE = EPT // STAGE
SPAIR = STAGE // CHUNK // 2
NPAD = 10240
NPT = NPAD // NSUB
ZROWS = 128

_f32 = jnp.float32
_i32 = jnp.int32

_S_BLK = np.repeat(np.eye(H1, dtype=np.float32), HID_C, axis=0)
_E8 = _S_BLK.T.copy()
_M16 = np.zeros((HC, HID_C), dtype=np.float32)
for _j in range(HC):
    _M16[_j, _j % HID_C] = 1.0 / H1
_ONEHOT16 = np.eye(16, dtype=np.float32)


def _isplat(i):
    return lax.iota(_i32, 16) * 0 + i


def _zeros16():
    return lax.broadcast_in_dim(_f32(0.0), (16,), ())


def _splat(vec, i):
    idx = _isplat(i)
    return lax.gather(
        vec, idx[:, None],
        lax.GatherDimensionNumbers(offset_dims=(), collapsed_slice_dims=(0,),
                                   start_index_map=(0,)),
        (1,), mode=lax.GatherScatterMode.PROMISE_IN_BOUNDS)


def _copy_idx(srcref, off, dstref):
    for o in (0, 16, 24):
        dstref[pl.ds(o, 16)] = srcref[pl.ds(off + o, 16)]

def _lrelu_exp(z):
    return jnp.exp(jnp.maximum(z, 0.2 * z))


def _prep1_body(x_ref, w1_ref, asrcm_ref, adstm_ref, ear_ref,
                g1_ref, asrct_ref, adstt_ref, mean_ref):
    h1 = jnp.dot(x_ref[...], w1_ref[...], preferred_element_type=_f32)
    a_src = jnp.dot(h1, asrcm_ref[...], preferred_element_type=_f32)
    a_dst = jnp.dot(h1, adstm_ref[...], preferred_element_type=_f32)
    z8 = jnp.zeros((N, 8), _f32)
    g1_ref[...] = h1
    asrct_ref[...] = jnp.concatenate([a_src, z8], axis=1)
    adstt_ref[...] = jnp.concatenate([a_dst, z8], axis=1)
    mean_ref[...] = (jnp.sum(ear_ref[...]) / E).reshape(1, 1)


BR = 2000


def _post1_body(accm_ref, accw_ref, g1_ref, asrct_ref, adstt_ref,
                mc1_ref, bias1_ref,
                e8_ref, m16_ref, w2_ref, asrc2m_ref, adst2m_ref, mc2_ref,
                g2_ref, adst2t_ref, wself2_ref):
    msum = accm_ref[0] + accm_ref[1]
    wsum = accw_ref[0] + accw_ref[1]
    h1 = g1_ref[...]
    a_src = asrct_ref[:, :8]
    a_dst = adstt_ref[:, :8]
    wself = _lrelu_exp(a_src + a_dst + mc1_ref[...])
    w128 = jnp.dot(wself, e8_ref[...], preferred_element_type=_f32)
    s128 = jnp.dot(wsum[:, :8] + wself, e8_ref[...],
                   preferred_element_type=_f32)
    msg = msum + w128 * h1
    pre = msg / s128
    out1 = jnp.maximum(
        jnp.dot(pre, m16_ref[...], preferred_element_type=_f32) + bias1_ref[...],
        0.0)
    h2 = jnp.dot(out1, w2_ref[...], preferred_element_type=_f32)
    a2s = jnp.dot(h2, asrc2m_ref[...], preferred_element_type=_f32)
    a2d = jnp.dot(h2, adst2m_ref[...], preferred_element_type=_f32)
    z2 = a2s + a2d + mc2_ref[...]
    wself2 = _lrelu_exp(z2)
    ones1 = jnp.ones((BR, 1), _f32)
    g2_ref[...] = jnp.concatenate(
        [h2, ones1, a2s[:, 0:1], jnp.zeros((BR, 6), _f32)], axis=1)
    adst2t_ref[...] = jnp.broadcast_to(a2d[:, 0:1], (BR, 16))
    wself2_ref[...] = jnp.broadcast_to(wself2[:, 0:1], (BR, 16))


def _post2_body(acc2_ref, g2_ref, wself2_ref, bias2_ref, out_ref):
    tot = acc2_ref[0, :N] + acc2_ref[1, :N] + wself2_ref[...] * g2_ref[...]
    s = tot[:, OUT_C:OUT_C + 1]
    out_ref[...] = tot[:, :OUT_C] / s + bias2_ref[...]


_MESH = plsc.VectorSubcoreMesh(core_axis_name="c", subcore_axis_name="s")


@functools.partial(
    pl.kernel,
    mesh=_MESH,
    compiler_params=pltpu.CompilerParams(use_tc_tiling_on_sc=False),
    out_type=(
        jax.ShapeDtypeStruct((NCORES, NPAD, HC), _f32),
        jax.ShapeDtypeStruct((NCORES, NPAD, 16), _f32),
    ),
    scratch_types=[
        pltpu.VMEM((STAGE,), _i32),
        pltpu.VMEM((STAGE,), _i32),
        pltpu.VMEM((STAGE,), _f32),
        [pltpu.VMEM((CHUNK, HC), _f32)] * 2,
        [pltpu.VMEM((CHUNK, 16), _f32)] * 2,
        [pltpu.VMEM((CHUNK, 16), _f32)] * 2,
        [pltpu.VMEM((CHUNK, 16), _f32)] * 2,
        pltpu.VMEM((16,), _f32),
        [pltpu.VMEM((CHUNK,), _i32)] * 2,
        [pltpu.VMEM((CHUNK,), _i32)] * 2,
        pltpu.VMEM_SHARED((NPAD, HC), _f32),
        pltpu.VMEM_SHARED((NPAD, 16), _f32),
        [pltpu.SemaphoreType.DMA] * 6,
        [pltpu.SemaphoreType.DMA] * 4,
    ],
)
def _edges1(g1, asrct, adstt, src_a, dst_a, ea_a, c1_a, zm_a, zw_a, outm, outw,
            srcf, dstf2, eaf, rows2, asrcr2, adrows2, wbuf2, c1v, srcc2, dstc2,
            accm, accw, gsem, ssem):
    cc = lax.axis_index("c")
    ss = lax.axis_index("s")
    row0 = ss * NPT
    pltpu.sync_copy(zm_a, accm.at[pl.ds(row0, NPT)])
    pltpu.sync_copy(zw_a, accw.at[pl.ds(row0, NPT)])
    pltpu.sync_copy(c1_a, c1v)
    plsc.subcore_barrier()

    c1vec = c1v[...]
    ebase = cc * (E // NCORES) + ss * EPT

    def fire_gathers(lt, b):
        _copy_idx(srcf, lt * CHUNK, srcc2[b])
        _copy_idx(dstf2, lt * CHUNK, dstc2[b])
        pltpu.async_copy(g1.at[srcc2[b]], rows2[b], gsem[3 * b])
        pltpu.async_copy(asrct.at[srcc2[b]], asrcr2[b], gsem[3 * b + 1])
        pltpu.async_copy(adstt.at[dstc2[b]], adrows2[b], gsem[3 * b + 2])

    def wait_gathers(lt, b):
        pltpu.make_async_copy(g1.at[srcc2[b]], rows2[b], gsem[3 * b]).wait()
        pltpu.make_async_copy(asrct.at[srcc2[b]], asrcr2[b],
                              gsem[3 * b + 1]).wait()
        pltpu.make_async_copy(adstt.at[dstc2[b]], adrows2[b],
                              gsem[3 * b + 2]).wait()

    def fire_scatters(lt, b):
        pltpu.async_copy(rows2[b], accm.at[dstc2[b]], ssem[2 * b], add=True)
        pltpu.async_copy(wbuf2[b], accw.at[dstc2[b]], ssem[2 * b + 1], add=True)

    def wait_scatters(lt, b):
        pltpu.make_async_copy(rows2[b], accm.at[dstc2[b]], ssem[2 * b]).wait()
        pltpu.make_async_copy(wbuf2[b], accw.at[dstc2[b]],
                              ssem[2 * b + 1]).wait()

    def compute(lt, b):
        rows, asrcr, adrows, wbuf = rows2[b], asrcr2[b], adrows2[b], wbuf2[b]
        for off, lo in ((0, 0), (16, 0), (24, 8)):
            ea16 = eaf[pl.ds(lt * CHUNK + off, 16)]
            for e in range(lo, 16):
                ei = off + e
                w = _lrelu_exp(asrcr[ei, :] + adrows[ei, :]
                               + _splat(ea16, e) * c1vec)
                wbuf[ei, :] = w
                for h in range(H1):
                    wh = _splat(w, h)
                    rows[ei, pl.ds(16 * h, 16)] = rows[ei, pl.ds(16 * h, 16)] * wh

    def stage_body(s, carry):
        @pl.when(s > 0)
        def _():
            wait_scatters(STAGE // CHUNK - 1, 1)
        sb = ebase + s * STAGE
        pltpu.sync_copy(src_a.at[pl.ds(sb, STAGE)], srcf)
        pltpu.sync_copy(dst_a.at[pl.ds(sb, STAGE)], dstf2)
        pltpu.sync_copy(ea_a.at[pl.ds(sb, STAGE)], eaf)
        fire_gathers(0, 0)

        def pair_body(j, pcarry):
            la = 2 * j
            wait_gathers(la, 0)
            compute(la, 0)

            @pl.when(j > 0)
            def _():
                wait_scatters(la - 1, 1)
            fire_scatters(la, 0)
            fire_gathers(la + 1, 1)

            wait_gathers(la + 1, 1)
            compute(la + 1, 1)
            wait_scatters(la, 0)
            fire_scatters(la + 1, 1)

            @pl.when(j < SPAIR - 1)
            def _():
                fire_gathers(la + 2, 0)
            return pcarry
        lax.fori_loop(0, SPAIR, pair_body, 0)
        return carry
    lax.fori_loop(0, NSTAGE, stage_body, 0)
    wait_scatters(STAGE // CHUNK - 1, 1)

    plsc.subcore_barrier()
    pltpu.sync_copy(accm.at[pl.ds(row0, NPT)], outm.at[cc, pl.ds(row0, NPT)])
    pltpu.sync_copy(accw.at[pl.ds(row0, NPT)], outw.at[cc, pl.ds(row0, NPT)])


@functools.partial(
    pl.kernel,
    mesh=_MESH,
    compiler_params=pltpu.CompilerParams(use_tc_tiling_on_sc=False),
    out_type=jax.ShapeDtypeStruct((NCORES, NPAD, ROWW2), _f32),
    scratch_types=[
        pltpu.VMEM((STAGE,), _i32),
        pltpu.VMEM((STAGE,), _i32),
        pltpu.VMEM((STAGE,), _f32),
        [pltpu.VMEM((CHUNK, ROWW2), _f32)] * 2,
        [pltpu.VMEM((CHUNK, 16), _f32)] * 2,
        pltpu.VMEM((16,), _f32),
        [pltpu.VMEM((CHUNK,), _i32)] * 2,
        [pltpu.VMEM((CHUNK,), _i32)] * 2,
        pltpu.VMEM_SHARED((NPAD, ROWW2), _f32),
        [pltpu.SemaphoreType.DMA] * 4,
        [pltpu.SemaphoreType.DMA] * 2,
    ],
)
def _edges2(g2, adst2t, src_a, dst_a, ea_a, c2_a, zw_a, out,
            srcf, dstf2, eaf, rows2, adrows2, c2v, srcc2, dstc2, acc, gsem, ssem):
    cc = lax.axis_index("c")
    ss = lax.axis_index("s")
    row0 = ss * NPT
    pltpu.sync_copy(zw_a, acc.at[pl.ds(row0, NPT)])
    pltpu.sync_copy(c2_a, c2v)
    plsc.subcore_barrier()

    c2vec = c2v[...]
    ebase = cc * (E // NCORES) + ss * EPT

    def fire_gathers(lt, b):
        _copy_idx(srcf, lt * CHUNK, srcc2[b])
        _copy_idx(dstf2, lt * CHUNK, dstc2[b])
        pltpu.async_copy(g2.at[srcc2[b]], rows2[b], gsem[2 * b])
        pltpu.async_copy(adst2t.at[dstc2[b]], adrows2[b], gsem[2 * b + 1])

    def wait_gathers(lt, b):
        pltpu.make_async_copy(g2.at[srcc2[b]], rows2[b], gsem[2 * b]).wait()
        pltpu.make_async_copy(adst2t.at[dstc2[b]], adrows2[b],
                              gsem[2 * b + 1]).wait()

    def fire_scatters(lt, b):
        pltpu.async_copy(rows2[b], acc.at[dstc2[b]], ssem[b], add=True)

    def wait_scatters(lt, b):
        pltpu.make_async_copy(rows2[b], acc.at[dstc2[b]], ssem[b]).wait()

    def compute(lt, b):
        rows, adrows = rows2[b], adrows2[b]
        for off, lo in ((0, 0), (16, 0), (24, 8)):
            ea16 = eaf[pl.ds(lt * CHUNK + off, 16)]
            for e in range(lo, 16):
                ei = off + e
                r = rows[ei, :]
                z = _splat(r, 9) + adrows[ei, :] + _splat(ea16, e) * c2vec
                rows[ei, :] = r * _lrelu_exp(z)

    def stage_body(s, carry):
        @pl.when(s > 0)
        def _():
            wait_scatters(STAGE // CHUNK - 1, 1)
        sb = ebase + s * STAGE
        pltpu.sync_copy(src_a.at[pl.ds(sb, STAGE)], srcf)
        pltpu.sync_copy(dst_a.at[pl.ds(sb, STAGE)], dstf2)
        pltpu.sync_copy(ea_a.at[pl.ds(sb, STAGE)], eaf)
        fire_gathers(0, 0)

        def pair_body(j, pcarry):
            la = 2 * j
            wait_gathers(la, 0)
            compute(la, 0)

            @pl.when(j > 0)
            def _():
                wait_scatters(la - 1, 1)
            fire_scatters(la, 0)
            fire_gathers(la + 1, 1)

            wait_gathers(la + 1, 1)
            compute(la + 1, 1)
            wait_scatters(la, 0)
            fire_scatters(la + 1, 1)

            @pl.when(j < SPAIR - 1)
            def _():
                fire_gathers(la + 2, 0)
            return pcarry
        lax.fori_loop(0, SPAIR, pair_body, 0)
        return carry
    lax.fori_loop(0, NSTAGE, stage_body, 0)
    wait_scatters(STAGE // CHUNK - 1, 1)

    plsc.subcore_barrier()
    pltpu.sync_copy(acc.at[pl.ds(row0, NPT)], out.at[cc, pl.ds(row0, NPT)])


def kernel(x, edge_index, edge_attr,
           W1, att_src1, att_dst1, W_edge1, att_edge1, bias1,
           W2, att_src2, att_dst2, W_edge2, att_edge2, bias2):
    src = edge_index[0].astype(_i32)
    dst = edge_index[1].astype(_i32)
    ea = edge_attr[:, 0]

    asrc1m = att_src1.reshape(HC, 1) * _S_BLK
    adst1m = att_dst1.reshape(HC, 1) * _S_BLK
    c1 = jnp.sum(W_edge1.reshape(H1, HID_C) * att_edge1[0], axis=-1)
    c1pad = jnp.concatenate([c1, jnp.zeros((8,), _f32)])
    c2 = jnp.sum(W_edge2.reshape(OUT_C) * att_edge2.reshape(OUT_C))
    c2vec = jnp.full((16,), c2, dtype=_f32)
    asrc2m = att_src2.reshape(OUT_C, 1) * np.eye(OUT_C, 16, dtype=np.float32)
    adst2m = att_dst2.reshape(OUT_C, 1) * np.eye(OUT_C, 16, dtype=np.float32)

    g1, asrct, adstt, mean_s = pl.pallas_call(
        _prep1_body,
        out_shape=(
            jax.ShapeDtypeStruct((N, HC), _f32),
            jax.ShapeDtypeStruct((N, 16), _f32),
            jax.ShapeDtypeStruct((N, 16), _f32),
            jax.ShapeDtypeStruct((1, 1), _f32),
        ),
    )(x, W1, asrc1m, adst1m, edge_attr.reshape(2500, 128))

    zm = jnp.zeros((NPT, HC), _f32)
    zw = jnp.zeros((NPT, 16), _f32)
    accm, accw = _edges1(g1, asrct, adstt, src, dst, ea, c1pad, zm, zw)

    mc1 = mean_s[0, 0] * c1.reshape(1, 8)
    mc2 = jnp.broadcast_to(mean_s[0, 0] * c2, (1, 16))

    _full = lambda bs: pl.BlockSpec(bs, lambda i: (0,) * len(bs))
    g2, adst2t, wself2 = pl.pallas_call(
        _post1_body,
        grid=(N // BR,),
        in_specs=[
            pl.BlockSpec((2, BR, HC), lambda i: (0, i, 0)),
            pl.BlockSpec((2, BR, 16), lambda i: (0, i, 0)),
            pl.BlockSpec((BR, HC), lambda i: (i, 0)),
            pl.BlockSpec((BR, 16), lambda i: (i, 0)),
            pl.BlockSpec((BR, 16), lambda i: (i, 0)),
            _full((1, 8)), _full((1, 16)), _full((8, HC)), _full((HC, 16)),
            _full((16, 8)), _full((8, 16)), _full((8, 16)), _full((1, 16)),
        ],
        out_specs=[pl.BlockSpec((BR, 16), lambda i: (i, 0))] * 3,
        out_shape=(
            jax.ShapeDtypeStruct((N, 16), _f32),
            jax.ShapeDtypeStruct((N, 16), _f32),
            jax.ShapeDtypeStruct((N, 16), _f32),
        ),
    )(accm, accw, g1, asrct, adstt, mc1, bias1.reshape(1, HID_C),
      jnp.asarray(_E8), jnp.asarray(_M16), W2, asrc2m, adst2m, mc2)

    acc2 = _edges2(g2, adst2t, src, dst, ea, c2vec, zw)

    out = pl.pallas_call(
        _post2_body,
        out_shape=jax.ShapeDtypeStruct((N, OUT_C), _f32),
    )(acc2, g2, wself2, bias2.reshape(1, OUT_C))
    return out
